# baseline (speedup 1.0000x reference)
import functools

import jax
import jax.numpy as jnp
from jax import lax
from jax.experimental import pallas as pl
from jax.experimental.pallas import tpu as pltpu
from jax.experimental.pallas import tpu_sc as plsc


def kernel(W):
    mesh = plsc.VectorSubcoreMesh(core_axis_name="c", subcore_axis_name="s")

    @functools.partial(
        pl.kernel,
        mesh=mesh,
        out_type=jax.ShapeDtypeStruct((1, W.shape[1]), W.dtype),
    )
    def _lookup(w_hbm, out_hbm):
        first = (lax.axis_index("c") == 0) & (lax.axis_index("s") == 0)

        @pl.when(first)
        def _():
            pltpu.sync_copy(w_hbm.at[pl.ds(0, 1)], out_hbm)

    return _lookup(W)


# speedup vs baseline: 1.1806x; 1.1806x over previous
import functools

import jax
import jax.numpy as jnp
from jax import lax
from jax.experimental import pallas as pl
from jax.experimental.pallas import tpu as pltpu
from jax.experimental.pallas import tpu_sc as plsc


def kernel(W):
    mesh = plsc.ScalarSubcoreMesh(axis_name="c", num_cores=1)

    @functools.partial(
        pl.kernel,
        mesh=mesh,
        out_type=jax.ShapeDtypeStruct((1, W.shape[1]), W.dtype),
    )
    def _lookup(w_hbm, out_hbm):
        pltpu.sync_copy(w_hbm.at[pl.ds(0, 1)], out_hbm)

    return _lookup(W)


# speedup vs baseline: 15.9998x; 13.5518x over previous
import jax
import jax.numpy as jnp
from jax.experimental import pallas as pl


def _body(w_ref, o_ref):
    o_ref[...] = w_ref[0:1, :]


def kernel(W):
    return pl.pallas_call(
        _body,
        grid=(1,),
        in_specs=[pl.BlockSpec((8, W.shape[1]), lambda i: (0, 0))],
        out_specs=pl.BlockSpec((1, W.shape[1]), lambda i: (0, 0)),
        out_shape=jax.ShapeDtypeStruct((1, W.shape[1]), W.dtype),
    )(W)


# speedup vs baseline: 18.3565x; 1.1473x over previous
import jax
import jax.numpy as jnp
from jax.experimental import pallas as pl
from jax.experimental.pallas import tpu as pltpu


def _body(w_hbm, o_hbm, sem):
    pltpu.make_async_copy(w_hbm.at[pl.ds(0, 1)], o_hbm, sem).start()
    pltpu.make_async_copy(w_hbm.at[pl.ds(0, 1)], o_hbm, sem).wait()


def kernel(W):
    return pl.pallas_call(
        _body,
        in_specs=[pl.BlockSpec(memory_space=pltpu.MemorySpace.HBM)],
        out_specs=pl.BlockSpec(memory_space=pltpu.MemorySpace.HBM),
        out_shape=jax.ShapeDtypeStruct((1, W.shape[1]), W.dtype),
        scratch_shapes=[pltpu.SemaphoreType.DMA],
    )(W)
